# baseline (speedup 1.0000x reference)
import dataclasses
import functools

import jax
import jax.numpy as jnp
from jax import lax
from jax.experimental import pallas as pl
from jax.experimental.pallas import tpu as pltpu
from jax.experimental.pallas import tpu_sc as plsc

EMB_D = 32
BATCH = 16384
NC = 2
NS = 16
LANES = 16
NW = NC * NS
BPW = BATCH // NW


def _sc_partials(u_idx, i_idx, r, uY, iY):
    mesh = plsc.VectorSubcoreMesh(core_axis_name="c", subcore_axis_name="s")
    cp = pltpu.CompilerParams()
    if "needs_layout_passes" in pltpu.CompilerParams.__dataclass_fields__:
        cp = dataclasses.replace(cp, needs_layout_passes=False)
    if "use_tc_tiling_on_sc" in pltpu.CompilerParams.__dataclass_fields__:
        cp = dataclasses.replace(cp, use_tc_tiling_on_sc=False)

    @functools.partial(
        pl.kernel,
        mesh=mesh,
        compiler_params=cp,
        out_type=jax.ShapeDtypeStruct((NW, LANES), jnp.float32),
        scratch_types=[
            pltpu.VMEM((BPW,), jnp.int32),
            pltpu.VMEM((BPW,), jnp.int32),
            pltpu.VMEM((BPW,), jnp.float32),
            pltpu.VMEM((BPW, EMB_D), jnp.float32),
            pltpu.VMEM((BPW, EMB_D), jnp.float32),
            pltpu.VMEM((LANES,), jnp.float32),
            pltpu.SemaphoreType.DMA,
            pltpu.SemaphoreType.DMA,
        ],
    )
    def kern(u_hbm, i_hbm, r_hbm, uY_hbm, iY_hbm, out_hbm,
             u_v, i_v, r_v, lu_v, li_v, sq_v, sem_u, sem_i):
        wid = lax.axis_index("s") * NC + lax.axis_index("c")
        base = wid * BPW

        pltpu.sync_copy(u_hbm.at[pl.ds(base, BPW)], u_v)
        pltpu.sync_copy(i_hbm.at[pl.ds(base, BPW)], i_v)
        cp_u = pltpu.async_copy(uY_hbm.at[u_v], lu_v, sem_u)
        cp_i = pltpu.async_copy(iY_hbm.at[i_v], li_v, sem_i)
        pltpu.sync_copy(r_hbm.at[pl.ds(base, BPW)], r_v)
        cp_u.wait()
        cp_i.wait()

        sq_v[...] = jnp.zeros((LANES,), jnp.float32)

        @pl.loop(0, BPW, step=LANES)
        def _(jb):
            rows = jb + lax.iota(jnp.int32, LANES)
            acc = jnp.zeros((LANES,), jnp.float32)
            for d in range(EMB_D):
                col = jnp.full((LANES,), d, jnp.int32)
                a = plsc.load_gather(lu_v, [rows, col])
                b = plsc.load_gather(li_v, [rows, col])
                acc = acc + a * b
            err = r_v[pl.ds(jb, LANES)] - acc
            sq_v[...] = sq_v[...] + err * err

        pltpu.sync_copy(sq_v, out_hbm.at[wid])

    return kern(u_idx, i_idx, r, uY, iY)


def _tc_mean(partials):
    def body(p_ref, o_ref):
        o_ref[0, 0] = jnp.sum(p_ref[...]) * (1.0 / BATCH)

    out = pl.pallas_call(
        body,
        out_shape=jax.ShapeDtypeStruct((1, 1), jnp.float32),
        out_specs=pl.BlockSpec(memory_space=pltpu.SMEM),
    )(partials)
    return out[0, 0]


@jax.jit
def _mf_loss(interaction, uY, iY):
    u = interaction[:, 0].astype(jnp.int32)
    i = interaction[:, 1].astype(jnp.int32)
    r = interaction[:, 2].astype(jnp.float32)
    partials = _sc_partials(u, i, r, uY, iY)
    return _tc_mean(partials)


def kernel(interaction, uY, iY):
    return _mf_loss(interaction, uY, iY)


# speedup vs baseline: 1.4481x; 1.4481x over previous
import dataclasses
import functools

import jax
import jax.numpy as jnp
from jax import lax
from jax.experimental import pallas as pl
from jax.experimental.pallas import tpu as pltpu
from jax.experimental.pallas import tpu_sc as plsc

NU = 1000000
EMB_D = 32
BATCH = 16384
NC = 2
NS = 16
LANES = 16
NW = NC * NS
BPW = BATCH // NW
CHUNK = 256
TBLK = 4096


GROUP = 4
GROUP_W = GROUP * EMB_D


QROW = TBLK // GROUP
NBLK = -(-NU // TBLK)
NROWS = NBLK * QROW


def _tc_stage(tY_t):
    def body(x_ref, o_ref):
        vt = jnp.transpose(x_ref[...])
        parts = [
            lax.slice(vt, (QROW * p, 0), (QROW * (p + 1), EMB_D))
            for p in range(GROUP)
        ]
        o_ref[...] = jnp.concatenate(parts, axis=1)

    return pl.pallas_call(
        body,
        grid=(NBLK,),
        in_specs=[pl.BlockSpec((EMB_D, TBLK), lambda g: (0, g))],
        out_specs=pl.BlockSpec((QROW, GROUP_W), lambda g: (g, 0)),
        out_shape=jax.ShapeDtypeStruct((NROWS, GROUP_W), jnp.float32),
        compiler_params=pltpu.CompilerParams(
            dimension_semantics=("arbitrary",)),
    )(tY_t)


def _sc_partials(u_idx, i_idx, r, uY, iY):
    mesh = plsc.VectorSubcoreMesh(core_axis_name="c", subcore_axis_name="s")
    cp = pltpu.CompilerParams()
    if "needs_layout_passes" in pltpu.CompilerParams.__dataclass_fields__:
        cp = dataclasses.replace(cp, needs_layout_passes=False)

    @functools.partial(
        pl.kernel,
        mesh=mesh,
        compiler_params=cp,
        out_type=jax.ShapeDtypeStruct((NW, LANES), jnp.float32),
        scratch_types=[
            pltpu.VMEM((BPW,), jnp.int32),
            pltpu.VMEM((BPW,), jnp.int32),
            pltpu.VMEM((BPW,), jnp.float32),
            pltpu.VMEM((BPW,), jnp.int32),
            pltpu.VMEM((BPW,), jnp.int32),
            pltpu.VMEM((CHUNK, GROUP_W), jnp.float32),
            pltpu.VMEM((CHUNK, GROUP_W), jnp.float32),
            pltpu.VMEM((LANES,), jnp.float32),
            pltpu.SemaphoreType.DMA,
            pltpu.SemaphoreType.DMA,
        ],
    )
    def kern(u_hbm, i_hbm, r_hbm, uY_hbm, iY_hbm, out_hbm,
             u_v, i_v, r_v, du_v, di_v, lu_v, li_v, sq_v, sem_u, sem_i):
        wid = lax.axis_index("s") * NC + lax.axis_index("c")
        base = wid * BPW

        pltpu.sync_copy(u_hbm.at[pl.ds(base, BPW)], u_v)
        pltpu.sync_copy(i_hbm.at[pl.ds(base, BPW)], i_v)
        pltpu.sync_copy(r_hbm.at[pl.ds(base, BPW)], r_v)

        def staged_row(ids):
            return lax.shift_left(
                lax.shift_right_logical(ids, 12), 10) | (ids & 1023)

        @pl.loop(0, BPW, step=LANES)
        def _(k):
            du_v[pl.ds(k, LANES)] = staged_row(u_v[pl.ds(k, LANES)])
            di_v[pl.ds(k, LANES)] = staged_row(i_v[pl.ds(k, LANES)])

        sq_v[...] = jnp.zeros((LANES,), jnp.float32)

        for c in range(BPW // CHUNK):
            cbase = c * CHUNK
            cp_u = pltpu.async_copy(
                uY_hbm.at[du_v.at[pl.ds(cbase, CHUNK)]], lu_v, sem_u)
            cp_i = pltpu.async_copy(
                iY_hbm.at[di_v.at[pl.ds(cbase, CHUNK)]], li_v, sem_i)
            cp_u.wait()
            cp_i.wait()

            @pl.loop(0, CHUNK, step=LANES)
            def _(jb):
                rows = jb + lax.iota(jnp.int32, LANES)
                three = jnp.full((LANES,), 3, jnp.int32)
                cu = (lax.shift_right_logical(
                    u_v[pl.ds(cbase + jb, LANES)], 10) & three) * EMB_D
                ci = (lax.shift_right_logical(
                    i_v[pl.ds(cbase + jb, LANES)], 10) & three) * EMB_D
                acc = jnp.zeros((LANES,), jnp.float32)
                for d in range(EMB_D):
                    a = plsc.load_gather(lu_v, [rows, cu + d])
                    b = plsc.load_gather(li_v, [rows, ci + d])
                    acc = acc + a * b
                err = r_v[pl.ds(cbase + jb, LANES)] - acc
                sq_v[...] = sq_v[...] + err * err

        pltpu.sync_copy(sq_v, out_hbm.at[wid])

    return kern(u_idx, i_idx, r, uY, iY)


def _tc_mean(partials):
    def body(p_ref, o_ref):
        o_ref[0, 0] = jnp.sum(p_ref[...]) * (1.0 / BATCH)

    out = pl.pallas_call(
        body,
        out_shape=jax.ShapeDtypeStruct((1, 1), jnp.float32),
        out_specs=pl.BlockSpec(memory_space=pltpu.SMEM),
    )(partials)
    return out[0, 0]


@jax.jit
def _mf_loss(interaction, uY, iY):
    u = interaction[:, 0].astype(jnp.int32)
    i = interaction[:, 1].astype(jnp.int32)
    r = interaction[:, 2].astype(jnp.float32)
    uYs = _tc_stage(uY.T)
    iYs = _tc_stage(iY.T)
    partials = _sc_partials(u, i, r, uYs, iYs)
    return _tc_mean(partials)


def kernel(interaction, uY, iY):
    return _mf_loss(interaction, uY, iY)


# speedup vs baseline: 1.4485x; 1.0003x over previous
import dataclasses
import functools

import jax
import jax.numpy as jnp
from jax import lax
from jax.experimental import pallas as pl
from jax.experimental.pallas import tpu as pltpu
from jax.experimental.pallas import tpu_sc as plsc

NU = 1000000
EMB_D = 32
BATCH = 16384
NC = 2
NS = 16
LANES = 16
NW = NC * NS
BPW = BATCH // NW
CHUNK = 256
TBLK = 4096


GROUP = 4
GROUP_W = GROUP * EMB_D


QROW = TBLK // GROUP
NBLK = -(-NU // TBLK)
NROWS = NBLK * QROW


def _tc_stage(tY_t):
    def body(x_ref, o_ref):
        vt = jnp.transpose(x_ref[...])
        parts = [
            lax.slice(vt, (QROW * p, 0), (QROW * (p + 1), EMB_D))
            for p in range(GROUP)
        ]
        o_ref[...] = jnp.concatenate(parts, axis=1)

    return pl.pallas_call(
        body,
        grid=(NBLK,),
        in_specs=[pl.BlockSpec((EMB_D, TBLK), lambda g: (0, g))],
        out_specs=pl.BlockSpec((QROW, GROUP_W), lambda g: (g, 0)),
        out_shape=jax.ShapeDtypeStruct((NROWS, GROUP_W), jnp.float32),
        compiler_params=pltpu.CompilerParams(
            dimension_semantics=("parallel",)),
    )(tY_t)


def _sc_partials(u_idx, i_idx, r, uY, iY):
    mesh = plsc.VectorSubcoreMesh(core_axis_name="c", subcore_axis_name="s")
    cp = pltpu.CompilerParams()
    if "needs_layout_passes" in pltpu.CompilerParams.__dataclass_fields__:
        cp = dataclasses.replace(cp, needs_layout_passes=False)

    @functools.partial(
        pl.kernel,
        mesh=mesh,
        compiler_params=cp,
        out_type=jax.ShapeDtypeStruct((NW, LANES), jnp.float32),
        scratch_types=[
            pltpu.VMEM((BPW,), jnp.int32),
            pltpu.VMEM((BPW,), jnp.int32),
            pltpu.VMEM((BPW,), jnp.float32),
            pltpu.VMEM((BPW,), jnp.int32),
            pltpu.VMEM((BPW,), jnp.int32),
            pltpu.VMEM((CHUNK, GROUP_W), jnp.float32),
            pltpu.VMEM((CHUNK, GROUP_W), jnp.float32),
            pltpu.VMEM((LANES,), jnp.float32),
            pltpu.SemaphoreType.DMA,
            pltpu.SemaphoreType.DMA,
        ],
    )
    def kern(u_hbm, i_hbm, r_hbm, uY_hbm, iY_hbm, out_hbm,
             u_v, i_v, r_v, du_v, di_v, lu_v, li_v, sq_v, sem_u, sem_i):
        wid = lax.axis_index("s") * NC + lax.axis_index("c")
        base = wid * BPW

        pltpu.sync_copy(u_hbm.at[pl.ds(base, BPW)], u_v)
        pltpu.sync_copy(i_hbm.at[pl.ds(base, BPW)], i_v)
        pltpu.sync_copy(r_hbm.at[pl.ds(base, BPW)], r_v)

        def staged_row(ids):
            return lax.shift_left(
                lax.shift_right_logical(ids, 12), 10) | (ids & 1023)

        @pl.loop(0, BPW, step=LANES)
        def _(k):
            du_v[pl.ds(k, LANES)] = staged_row(u_v[pl.ds(k, LANES)])
            di_v[pl.ds(k, LANES)] = staged_row(i_v[pl.ds(k, LANES)])

        sq_v[...] = jnp.zeros((LANES,), jnp.float32)

        for c in range(BPW // CHUNK):
            cbase = c * CHUNK
            cp_u = pltpu.async_copy(
                uY_hbm.at[du_v.at[pl.ds(cbase, CHUNK)]], lu_v, sem_u)
            cp_i = pltpu.async_copy(
                iY_hbm.at[di_v.at[pl.ds(cbase, CHUNK)]], li_v, sem_i)
            cp_u.wait()
            cp_i.wait()

            @pl.loop(0, CHUNK, step=LANES)
            def _(jb):
                rows = jb + lax.iota(jnp.int32, LANES)
                three = jnp.full((LANES,), 3, jnp.int32)
                cu = (lax.shift_right_logical(
                    u_v[pl.ds(cbase + jb, LANES)], 10) & three) * EMB_D
                ci = (lax.shift_right_logical(
                    i_v[pl.ds(cbase + jb, LANES)], 10) & three) * EMB_D
                acc = jnp.zeros((LANES,), jnp.float32)
                for d in range(EMB_D):
                    a = plsc.load_gather(lu_v, [rows, cu + d])
                    b = plsc.load_gather(li_v, [rows, ci + d])
                    acc = acc + a * b
                err = r_v[pl.ds(cbase + jb, LANES)] - acc
                sq_v[...] = sq_v[...] + err * err

        pltpu.sync_copy(sq_v, out_hbm.at[wid])

    return kern(u_idx, i_idx, r, uY, iY)


def _tc_mean(partials):
    def body(p_ref, o_ref):
        o_ref[0, 0] = jnp.sum(p_ref[...]) * (1.0 / BATCH)

    out = pl.pallas_call(
        body,
        out_shape=jax.ShapeDtypeStruct((1, 1), jnp.float32),
        out_specs=pl.BlockSpec(memory_space=pltpu.SMEM),
    )(partials)
    return out[0, 0]


@jax.jit
def _mf_loss(interaction, uY, iY):
    u = interaction[:, 0].astype(jnp.int32)
    i = interaction[:, 1].astype(jnp.int32)
    r = interaction[:, 2].astype(jnp.float32)
    uYs = _tc_stage(uY.T)
    iYs = _tc_stage(iY.T)
    partials = _sc_partials(u, i, r, uYs, iYs)
    return _tc_mean(partials)


def kernel(interaction, uY, iY):
    return _mf_loss(interaction, uY, iY)


# speedup vs baseline: 2.1146x; 1.4598x over previous
import dataclasses
import functools

import jax
import jax.numpy as jnp
from jax import lax
from jax.experimental import pallas as pl
from jax.experimental.pallas import tpu as pltpu
from jax.experimental.pallas import tpu_sc as plsc

NU = 1000000
EMB_D = 32
BATCH = 16384
NC = 2
NS = 16
LANES = 16
NW = NC * NS
BPW = BATCH // NW
CHUNK = 256
TBLK = 4096


GROUP = 4
GROUP_W = GROUP * EMB_D


QROW = TBLK // GROUP
NBLK = -(-NU // TBLK)
NROWS = NBLK * QROW


def _tc_stage(tY_t):
    def body(x_ref, o_ref):
        x = x_ref[...]
        xs = jnp.concatenate(
            [lax.slice(x, (0, QROW * p), (EMB_D, QROW * (p + 1)))
             for p in range(GROUP)], axis=0)
        o_ref[...] = jnp.transpose(xs)

    return pl.pallas_call(
        body,
        grid=(NBLK,),
        in_specs=[pl.BlockSpec((EMB_D, TBLK), lambda g: (0, g))],
        out_specs=pl.BlockSpec((QROW, GROUP_W), lambda g: (g, 0)),
        out_shape=jax.ShapeDtypeStruct((NROWS, GROUP_W), jnp.float32),
        compiler_params=pltpu.CompilerParams(
            dimension_semantics=("parallel",)),
    )(tY_t)


def _sc_partials(u_idx, i_idx, r, uY, iY):
    mesh = plsc.VectorSubcoreMesh(core_axis_name="c", subcore_axis_name="s")
    cp = pltpu.CompilerParams()
    if "needs_layout_passes" in pltpu.CompilerParams.__dataclass_fields__:
        cp = dataclasses.replace(cp, needs_layout_passes=False)

    @functools.partial(
        pl.kernel,
        mesh=mesh,
        compiler_params=cp,
        out_type=jax.ShapeDtypeStruct((NW, LANES), jnp.float32),
        scratch_types=[
            pltpu.VMEM((BPW,), jnp.int32),
            pltpu.VMEM((BPW,), jnp.int32),
            pltpu.VMEM((BPW,), jnp.float32),
            pltpu.VMEM((BPW,), jnp.int32),
            pltpu.VMEM((BPW,), jnp.int32),
            pltpu.VMEM((CHUNK, GROUP_W), jnp.float32),
            pltpu.VMEM((CHUNK, GROUP_W), jnp.float32),
            pltpu.VMEM((LANES,), jnp.float32),
            pltpu.SemaphoreType.DMA,
            pltpu.SemaphoreType.DMA,
        ],
    )
    def kern(u_hbm, i_hbm, r_hbm, uY_hbm, iY_hbm, out_hbm,
             u_v, i_v, r_v, du_v, di_v, lu_v, li_v, sq_v, sem_u, sem_i):
        wid = lax.axis_index("s") * NC + lax.axis_index("c")
        base = wid * BPW

        pltpu.sync_copy(u_hbm.at[pl.ds(base, BPW)], u_v)
        pltpu.sync_copy(i_hbm.at[pl.ds(base, BPW)], i_v)
        pltpu.sync_copy(r_hbm.at[pl.ds(base, BPW)], r_v)

        def staged_row(ids):
            return lax.shift_left(
                lax.shift_right_logical(ids, 12), 10) | (ids & 1023)

        @pl.loop(0, BPW, step=LANES)
        def _(k):
            du_v[pl.ds(k, LANES)] = staged_row(u_v[pl.ds(k, LANES)])
            di_v[pl.ds(k, LANES)] = staged_row(i_v[pl.ds(k, LANES)])

        sq_v[...] = jnp.zeros((LANES,), jnp.float32)

        for c in range(BPW // CHUNK):
            cbase = c * CHUNK
            cp_u = pltpu.async_copy(
                uY_hbm.at[du_v.at[pl.ds(cbase, CHUNK)]], lu_v, sem_u)
            cp_i = pltpu.async_copy(
                iY_hbm.at[di_v.at[pl.ds(cbase, CHUNK)]], li_v, sem_i)
            cp_u.wait()
            cp_i.wait()

            @pl.loop(0, CHUNK, step=LANES)
            def _(jb):
                rows = jb + lax.iota(jnp.int32, LANES)
                three = jnp.full((LANES,), 3, jnp.int32)
                cu = (lax.shift_right_logical(
                    u_v[pl.ds(cbase + jb, LANES)], 10) & three) * EMB_D
                ci = (lax.shift_right_logical(
                    i_v[pl.ds(cbase + jb, LANES)], 10) & three) * EMB_D
                acc = jnp.zeros((LANES,), jnp.float32)
                for d in range(EMB_D):
                    a = plsc.load_gather(lu_v, [rows, cu + d])
                    b = plsc.load_gather(li_v, [rows, ci + d])
                    acc = acc + a * b
                err = r_v[pl.ds(cbase + jb, LANES)] - acc
                sq_v[...] = sq_v[...] + err * err

        pltpu.sync_copy(sq_v, out_hbm.at[wid])

    return kern(u_idx, i_idx, r, uY, iY)


def _tc_mean(partials):
    def body(p_ref, o_ref):
        o_ref[0, 0] = jnp.sum(p_ref[...]) * (1.0 / BATCH)

    out = pl.pallas_call(
        body,
        out_shape=jax.ShapeDtypeStruct((1, 1), jnp.float32),
        out_specs=pl.BlockSpec(memory_space=pltpu.SMEM),
    )(partials)
    return out[0, 0]


@jax.jit
def _mf_loss(interaction, uY, iY):
    u = interaction[:, 0].astype(jnp.int32)
    i = interaction[:, 1].astype(jnp.int32)
    r = interaction[:, 2].astype(jnp.float32)
    uYs = _tc_stage(uY.T)
    iYs = _tc_stage(iY.T)
    partials = _sc_partials(u, i, r, uYs, iYs)
    return _tc_mean(partials)


def kernel(interaction, uY, iY):
    return _mf_loss(interaction, uY, iY)


# speedup vs baseline: 3.7447x; 1.7709x over previous
import dataclasses
import functools

import jax
import jax.numpy as jnp
from jax import lax
from jax.experimental import pallas as pl
from jax.experimental.pallas import tpu as pltpu
from jax.experimental.pallas import tpu_sc as plsc

NU = 1000000
EMB_D = 32
BATCH = 16384
NC = 2
NS = 16
LANES = 16
NW = NC * NS
BPW = BATCH // NW
CHUNK = 256
TBLK_BITS = 14
TBLK = 1 << TBLK_BITS


GROUP = 4
GROUP_W = GROUP * EMB_D


QROW_BITS = TBLK_BITS - 2
QROW = TBLK // GROUP
NBLK = -(-NU // TBLK)
NROWS = NBLK * QROW


def _tc_stage(tY_t):
    def body(x_ref, o_ref):
        x = x_ref[...]
        xs = jnp.concatenate(
            [lax.slice(x, (0, QROW * p), (EMB_D, QROW * (p + 1)))
             for p in range(GROUP)], axis=0)
        o_ref[...] = jnp.transpose(xs)

    return pl.pallas_call(
        body,
        grid=(NBLK,),
        in_specs=[pl.BlockSpec((EMB_D, TBLK), lambda g: (0, g))],
        out_specs=pl.BlockSpec((QROW, GROUP_W), lambda g: (g, 0)),
        out_shape=jax.ShapeDtypeStruct((NROWS, GROUP_W), jnp.float32),
        compiler_params=pltpu.CompilerParams(
            dimension_semantics=("parallel",)),
    )(tY_t)


def _sc_partials(u_idx, i_idx, r, uY, iY):
    mesh = plsc.VectorSubcoreMesh(core_axis_name="c", subcore_axis_name="s")
    cp = pltpu.CompilerParams()
    if "needs_layout_passes" in pltpu.CompilerParams.__dataclass_fields__:
        cp = dataclasses.replace(cp, needs_layout_passes=False)

    @functools.partial(
        pl.kernel,
        mesh=mesh,
        compiler_params=cp,
        out_type=jax.ShapeDtypeStruct((NW, LANES), jnp.float32),
        scratch_types=[
            pltpu.VMEM((BPW,), jnp.int32),
            pltpu.VMEM((BPW,), jnp.int32),
            pltpu.VMEM((BPW,), jnp.float32),
            pltpu.VMEM((BPW,), jnp.int32),
            pltpu.VMEM((BPW,), jnp.int32),
            pltpu.VMEM((CHUNK, GROUP_W), jnp.float32),
            pltpu.VMEM((CHUNK, GROUP_W), jnp.float32),
            pltpu.VMEM((LANES,), jnp.float32),
            pltpu.SemaphoreType.DMA,
            pltpu.SemaphoreType.DMA,
        ],
    )
    def kern(u_hbm, i_hbm, r_hbm, uY_hbm, iY_hbm, out_hbm,
             u_v, i_v, r_v, du_v, di_v, lu_v, li_v, sq_v, sem_u, sem_i):
        wid = lax.axis_index("s") * NC + lax.axis_index("c")
        base = wid * BPW

        pltpu.sync_copy(u_hbm.at[pl.ds(base, BPW)], u_v)
        pltpu.sync_copy(i_hbm.at[pl.ds(base, BPW)], i_v)
        pltpu.sync_copy(r_hbm.at[pl.ds(base, BPW)], r_v)

        def staged_row(ids):
            return lax.shift_left(
                lax.shift_right_logical(ids, TBLK_BITS),
                QROW_BITS) | (ids & (QROW - 1))

        @pl.loop(0, BPW, step=LANES)
        def _(k):
            du_v[pl.ds(k, LANES)] = staged_row(u_v[pl.ds(k, LANES)])
            di_v[pl.ds(k, LANES)] = staged_row(i_v[pl.ds(k, LANES)])

        sq_v[...] = jnp.zeros((LANES,), jnp.float32)

        for c in range(BPW // CHUNK):
            cbase = c * CHUNK
            cp_u = pltpu.async_copy(
                uY_hbm.at[du_v.at[pl.ds(cbase, CHUNK)]], lu_v, sem_u)
            cp_i = pltpu.async_copy(
                iY_hbm.at[di_v.at[pl.ds(cbase, CHUNK)]], li_v, sem_i)
            cp_u.wait()
            cp_i.wait()

            @pl.loop(0, CHUNK, step=LANES)
            def _(jb):
                rows = jb + lax.iota(jnp.int32, LANES)
                three = jnp.full((LANES,), 3, jnp.int32)
                cu = (lax.shift_right_logical(
                    u_v[pl.ds(cbase + jb, LANES)], QROW_BITS) & three) * EMB_D
                ci = (lax.shift_right_logical(
                    i_v[pl.ds(cbase + jb, LANES)], QROW_BITS) & three) * EMB_D
                acc = jnp.zeros((LANES,), jnp.float32)
                for d in range(EMB_D):
                    a = plsc.load_gather(lu_v, [rows, cu + d])
                    b = plsc.load_gather(li_v, [rows, ci + d])
                    acc = acc + a * b
                err = r_v[pl.ds(cbase + jb, LANES)] - acc
                sq_v[...] = sq_v[...] + err * err

        pltpu.sync_copy(sq_v, out_hbm.at[wid])

    return kern(u_idx, i_idx, r, uY, iY)


def _tc_mean(partials):
    def body(p_ref, o_ref):
        o_ref[0, 0] = jnp.sum(p_ref[...]) * (1.0 / BATCH)

    out = pl.pallas_call(
        body,
        out_shape=jax.ShapeDtypeStruct((1, 1), jnp.float32),
        out_specs=pl.BlockSpec(memory_space=pltpu.SMEM),
    )(partials)
    return out[0, 0]


@jax.jit
def _mf_loss(interaction, uY, iY):
    u = interaction[:, 0].astype(jnp.int32)
    i = interaction[:, 1].astype(jnp.int32)
    r = interaction[:, 2].astype(jnp.float32)
    uYs = _tc_stage(uY.T)
    iYs = _tc_stage(iY.T)
    partials = _sc_partials(u, i, r, uYs, iYs)
    return _tc_mean(partials)


def kernel(interaction, uY, iY):
    return _mf_loss(interaction, uY, iY)


# speedup vs baseline: 4.3124x; 1.1516x over previous
import dataclasses
import functools

import jax
import jax.numpy as jnp
from jax import lax
from jax.experimental import pallas as pl
from jax.experimental.pallas import tpu as pltpu
from jax.experimental.pallas import tpu_sc as plsc

NU = 1000000
EMB_D = 32
BATCH = 16384
NC = 2
NS = 16
LANES = 16
NW = NC * NS
BPW = BATCH // NW
CHUNK = 256
TBLK_BITS = 16
TBLK = 1 << TBLK_BITS


GROUP = 4
GROUP_W = GROUP * EMB_D


QROW_BITS = TBLK_BITS - 2
QROW = TBLK // GROUP
NBLK = -(-NU // TBLK)
NROWS = NBLK * QROW


def _tc_stage(tY_t):
    def body(x_ref, o_ref):
        x = x_ref[...]
        xs = jnp.concatenate(
            [lax.slice(x, (0, QROW * p), (EMB_D, QROW * (p + 1)))
             for p in range(GROUP)], axis=0)
        o_ref[...] = jnp.transpose(xs)

    return pl.pallas_call(
        body,
        grid=(NBLK,),
        in_specs=[pl.BlockSpec((EMB_D, TBLK), lambda g: (0, g))],
        out_specs=pl.BlockSpec((QROW, GROUP_W), lambda g: (g, 0)),
        out_shape=jax.ShapeDtypeStruct((NROWS, GROUP_W), jnp.float32),
        compiler_params=pltpu.CompilerParams(
            dimension_semantics=("parallel",)),
    )(tY_t)


def _sc_partials(u_idx, i_idx, r, uY, iY):
    mesh = plsc.VectorSubcoreMesh(core_axis_name="c", subcore_axis_name="s")
    cp = pltpu.CompilerParams()
    if "needs_layout_passes" in pltpu.CompilerParams.__dataclass_fields__:
        cp = dataclasses.replace(cp, needs_layout_passes=False)

    @functools.partial(
        pl.kernel,
        mesh=mesh,
        compiler_params=cp,
        out_type=jax.ShapeDtypeStruct((NW, LANES), jnp.float32),
        scratch_types=[
            pltpu.VMEM((BPW,), jnp.int32),
            pltpu.VMEM((BPW,), jnp.int32),
            pltpu.VMEM((BPW,), jnp.float32),
            pltpu.VMEM((BPW,), jnp.int32),
            pltpu.VMEM((BPW,), jnp.int32),
            pltpu.VMEM((CHUNK, GROUP_W), jnp.float32),
            pltpu.VMEM((CHUNK, GROUP_W), jnp.float32),
            pltpu.VMEM((LANES,), jnp.float32),
            pltpu.SemaphoreType.DMA,
            pltpu.SemaphoreType.DMA,
        ],
    )
    def kern(u_hbm, i_hbm, r_hbm, uY_hbm, iY_hbm, out_hbm,
             u_v, i_v, r_v, du_v, di_v, lu_v, li_v, sq_v, sem_u, sem_i):
        wid = lax.axis_index("s") * NC + lax.axis_index("c")
        base = wid * BPW

        pltpu.sync_copy(u_hbm.at[pl.ds(base, BPW)], u_v)
        pltpu.sync_copy(i_hbm.at[pl.ds(base, BPW)], i_v)
        pltpu.sync_copy(r_hbm.at[pl.ds(base, BPW)], r_v)

        def staged_row(ids):
            return lax.shift_left(
                lax.shift_right_logical(ids, TBLK_BITS),
                QROW_BITS) | (ids & (QROW - 1))

        @pl.loop(0, BPW, step=LANES)
        def _(k):
            du_v[pl.ds(k, LANES)] = staged_row(u_v[pl.ds(k, LANES)])
            di_v[pl.ds(k, LANES)] = staged_row(i_v[pl.ds(k, LANES)])

        sq_v[...] = jnp.zeros((LANES,), jnp.float32)

        for c in range(BPW // CHUNK):
            cbase = c * CHUNK
            cp_u = pltpu.async_copy(
                uY_hbm.at[du_v.at[pl.ds(cbase, CHUNK)]], lu_v, sem_u)
            cp_i = pltpu.async_copy(
                iY_hbm.at[di_v.at[pl.ds(cbase, CHUNK)]], li_v, sem_i)
            cp_u.wait()
            cp_i.wait()

            @pl.loop(0, CHUNK, step=LANES)
            def _(jb):
                rows = jb + lax.iota(jnp.int32, LANES)
                three = jnp.full((LANES,), 3, jnp.int32)
                cu = (lax.shift_right_logical(
                    u_v[pl.ds(cbase + jb, LANES)], QROW_BITS) & three) * EMB_D
                ci = (lax.shift_right_logical(
                    i_v[pl.ds(cbase + jb, LANES)], QROW_BITS) & three) * EMB_D
                acc = jnp.zeros((LANES,), jnp.float32)
                for d in range(EMB_D):
                    a = plsc.load_gather(lu_v, [rows, cu + d])
                    b = plsc.load_gather(li_v, [rows, ci + d])
                    acc = acc + a * b
                err = r_v[pl.ds(cbase + jb, LANES)] - acc
                sq_v[...] = sq_v[...] + err * err

        pltpu.sync_copy(sq_v, out_hbm.at[wid])

    return kern(u_idx, i_idx, r, uY, iY)


def _tc_mean(partials):
    def body(p_ref, o_ref):
        o_ref[0, 0] = jnp.sum(p_ref[...]) * (1.0 / BATCH)

    out = pl.pallas_call(
        body,
        out_shape=jax.ShapeDtypeStruct((1, 1), jnp.float32),
        out_specs=pl.BlockSpec(memory_space=pltpu.SMEM),
    )(partials)
    return out[0, 0]


@jax.jit
def _mf_loss(interaction, uY, iY):
    u = interaction[:, 0].astype(jnp.int32)
    i = interaction[:, 1].astype(jnp.int32)
    r = interaction[:, 2].astype(jnp.float32)
    uYs = _tc_stage(uY.T)
    iYs = _tc_stage(iY.T)
    partials = _sc_partials(u, i, r, uYs, iYs)
    return _tc_mean(partials)


def kernel(interaction, uY, iY):
    return _mf_loss(interaction, uY, iY)


# speedup vs baseline: 5.0668x; 1.1749x over previous
import dataclasses
import functools

import jax
import jax.numpy as jnp
from jax import lax
from jax.experimental import pallas as pl
from jax.experimental.pallas import tpu as pltpu
from jax.experimental.pallas import tpu_sc as plsc

NU = 1000000
EMB_D = 32
BATCH = 16384
NC = 2
NS = 16
LANES = 16
NW = NC * NS
BPW = BATCH // NW
CHUNK = 256
TBLK_BITS = 16
TBLK = 1 << TBLK_BITS


GROUP = 8
GROUP_W = 128


QROW_BITS = TBLK_BITS - 3
QROW = TBLK // GROUP
NBLK = -(-NU // TBLK)
NROWS = NBLK * QROW


def _tc_stage(tY_t):
    def body(x_ref, o_ref):
        x = x_ref[...]
        xs = jnp.concatenate(
            [lax.slice(x, (0, QROW * p), (EMB_D, QROW * (p + 1)))
             for p in range(GROUP)], axis=0)
        vt = jnp.transpose(xs)
        bits = lax.bitcast_convert_type(vt, jnp.int32) + jnp.int32(0x8000)
        lo = lax.slice(bits, (0, 0), (QROW, GROUP_W))
        hi = lax.slice(bits, (0, GROUP_W), (QROW, 2 * GROUP_W))
        o_ref[...] = (lax.shift_right_logical(lo, 16)
                      | (hi & jnp.int32(-65536)))

    return pl.pallas_call(
        body,
        grid=(NBLK,),
        in_specs=[pl.BlockSpec((EMB_D, TBLK), lambda g: (0, g))],
        out_specs=pl.BlockSpec((QROW, GROUP_W), lambda g: (g, 0)),
        out_shape=jax.ShapeDtypeStruct((NROWS, GROUP_W), jnp.int32),
        compiler_params=pltpu.CompilerParams(
            dimension_semantics=("parallel",)),
    )(tY_t)


def _sc_partials(u_idx, i_idx, r, uY, iY):
    mesh = plsc.VectorSubcoreMesh(core_axis_name="c", subcore_axis_name="s")
    cp = pltpu.CompilerParams()
    if "needs_layout_passes" in pltpu.CompilerParams.__dataclass_fields__:
        cp = dataclasses.replace(cp, needs_layout_passes=False)

    @functools.partial(
        pl.kernel,
        mesh=mesh,
        compiler_params=cp,
        out_type=jax.ShapeDtypeStruct((NW, LANES), jnp.float32),
        scratch_types=[
            pltpu.VMEM((BPW,), jnp.int32),
            pltpu.VMEM((BPW,), jnp.int32),
            pltpu.VMEM((BPW,), jnp.float32),
            pltpu.VMEM((BPW,), jnp.int32),
            pltpu.VMEM((BPW,), jnp.int32),
            pltpu.VMEM((CHUNK, GROUP_W), jnp.int32),
            pltpu.VMEM((CHUNK, GROUP_W), jnp.int32),
            pltpu.VMEM((LANES,), jnp.float32),
            pltpu.SemaphoreType.DMA,
            pltpu.SemaphoreType.DMA,
        ],
    )
    def kern(u_hbm, i_hbm, r_hbm, uY_hbm, iY_hbm, out_hbm,
             u_v, i_v, r_v, du_v, di_v, lu_v, li_v, sq_v, sem_u, sem_i):
        wid = lax.axis_index("s") * NC + lax.axis_index("c")
        base = wid * BPW

        pltpu.sync_copy(u_hbm.at[pl.ds(base, BPW)], u_v)
        pltpu.sync_copy(i_hbm.at[pl.ds(base, BPW)], i_v)
        pltpu.sync_copy(r_hbm.at[pl.ds(base, BPW)], r_v)

        def staged_row(ids):
            return lax.shift_left(
                lax.shift_right_logical(ids, TBLK_BITS),
                QROW_BITS) | (ids & (QROW - 1))

        @pl.loop(0, BPW, step=LANES)
        def _(k):
            du_v[pl.ds(k, LANES)] = staged_row(u_v[pl.ds(k, LANES)])
            di_v[pl.ds(k, LANES)] = staged_row(i_v[pl.ds(k, LANES)])

        sq_v[...] = jnp.zeros((LANES,), jnp.float32)

        for c in range(BPW // CHUNK):
            cbase = c * CHUNK
            cp_u = pltpu.async_copy(
                uY_hbm.at[du_v.at[pl.ds(cbase, CHUNK)]], lu_v, sem_u)
            cp_i = pltpu.async_copy(
                iY_hbm.at[di_v.at[pl.ds(cbase, CHUNK)]], li_v, sem_i)
            cp_u.wait()
            cp_i.wait()

            @pl.loop(0, CHUNK, step=LANES)
            def _(jb):
                rows = jb + lax.iota(jnp.int32, LANES)
                three = jnp.full((LANES,), 3, jnp.int32)
                zero = jnp.zeros((LANES,), jnp.int32)
                pu = lax.shift_right_logical(
                    u_v[pl.ds(cbase + jb, LANES)], QROW_BITS)
                pi = lax.shift_right_logical(
                    i_v[pl.ds(cbase + jb, LANES)], QROW_BITS)
                cu = (pu & three) * EMB_D
                ci = (pi & three) * EMB_D
                selu = (lax.shift_right_logical(pu, 2) & three) > zero
                seli = (lax.shift_right_logical(pi, 2) & three) > zero
                acc = jnp.zeros((LANES,), jnp.float32)
                for d in range(EMB_D):
                    ua = plsc.load_gather(lu_v, [rows, cu + d])
                    ia = plsc.load_gather(li_v, [rows, ci + d])
                    u_lo, u_hi = plsc.unpack(
                        plsc.bitcast(ua, jnp.bfloat16),
                        format=plsc.PackFormat.INTERLEAVED)
                    i_lo, i_hi = plsc.unpack(
                        plsc.bitcast(ia, jnp.bfloat16),
                        format=plsc.PackFormat.INTERLEAVED)
                    uv = jnp.where(selu, u_hi, u_lo)
                    iv = jnp.where(seli, i_hi, i_lo)
                    acc = acc + uv * iv
                err = r_v[pl.ds(cbase + jb, LANES)] - acc
                sq_v[...] = sq_v[...] + err * err

        pltpu.sync_copy(sq_v, out_hbm.at[wid])

    return kern(u_idx, i_idx, r, uY, iY)


def _tc_mean(partials):
    def body(p_ref, o_ref):
        o_ref[0, 0] = jnp.sum(p_ref[...]) * (1.0 / BATCH)

    out = pl.pallas_call(
        body,
        out_shape=jax.ShapeDtypeStruct((1, 1), jnp.float32),
        out_specs=pl.BlockSpec(memory_space=pltpu.SMEM),
    )(partials)
    return out[0, 0]


@jax.jit
def _mf_loss(interaction, uY, iY):
    u = interaction[:, 0].astype(jnp.int32)
    i = interaction[:, 1].astype(jnp.int32)
    r = interaction[:, 2].astype(jnp.float32)
    uYs = _tc_stage(uY.T)
    iYs = _tc_stage(iY.T)
    partials = _sc_partials(u, i, r, uYs, iYs)
    return _tc_mean(partials)


def kernel(interaction, uY, iY):
    return _mf_loss(interaction, uY, iY)


# speedup vs baseline: 5.2297x; 1.0322x over previous
import dataclasses
import functools

import jax
import jax.numpy as jnp
from jax import lax
from jax.experimental import pallas as pl
from jax.experimental.pallas import tpu as pltpu
from jax.experimental.pallas import tpu_sc as plsc

NU = 1000000
EMB_D = 32
BATCH = 16384
NC = 2
NS = 16
LANES = 16
NW = NC * NS
BPW = BATCH // NW
CHUNK = 256
TBLK_BITS = 17
TBLK = 1 << TBLK_BITS


GROUP = 8
GROUP_W = 128


QROW_BITS = TBLK_BITS - 3
QROW = TBLK // GROUP
NBLK = -(-NU // TBLK)
NROWS = NBLK * QROW


def _tc_stage(tY_t):
    def body(x_ref, o_ref):
        x = x_ref[...]
        xs = jnp.concatenate(
            [lax.slice(x, (0, QROW * p), (EMB_D, QROW * (p + 1)))
             for p in range(GROUP)], axis=0)
        vt = jnp.transpose(xs)
        bits = lax.bitcast_convert_type(vt, jnp.int32) + jnp.int32(0x8000)
        lo = lax.slice(bits, (0, 0), (QROW, GROUP_W))
        hi = lax.slice(bits, (0, GROUP_W), (QROW, 2 * GROUP_W))
        o_ref[...] = (lax.shift_right_logical(lo, 16)
                      | (hi & jnp.int32(-65536)))

    return pl.pallas_call(
        body,
        grid=(NBLK,),
        in_specs=[pl.BlockSpec((EMB_D, TBLK), lambda g: (0, g))],
        out_specs=pl.BlockSpec((QROW, GROUP_W), lambda g: (g, 0)),
        out_shape=jax.ShapeDtypeStruct((NROWS, GROUP_W), jnp.int32),
        compiler_params=pltpu.CompilerParams(
            dimension_semantics=("parallel",)),
    )(tY_t)


def _sc_partials(u_idx, i_idx, r, uY, iY):
    mesh = plsc.VectorSubcoreMesh(core_axis_name="c", subcore_axis_name="s")
    cp = pltpu.CompilerParams()
    if "needs_layout_passes" in pltpu.CompilerParams.__dataclass_fields__:
        cp = dataclasses.replace(cp, needs_layout_passes=False)

    @functools.partial(
        pl.kernel,
        mesh=mesh,
        compiler_params=cp,
        out_type=jax.ShapeDtypeStruct((NW, LANES), jnp.float32),
        scratch_types=[
            pltpu.VMEM((BPW,), jnp.int32),
            pltpu.VMEM((BPW,), jnp.int32),
            pltpu.VMEM((BPW,), jnp.float32),
            pltpu.VMEM((BPW,), jnp.int32),
            pltpu.VMEM((BPW,), jnp.int32),
            pltpu.VMEM((CHUNK, GROUP_W), jnp.int32),
            pltpu.VMEM((CHUNK, GROUP_W), jnp.int32),
            pltpu.VMEM((LANES,), jnp.float32),
            pltpu.SemaphoreType.DMA,
            pltpu.SemaphoreType.DMA,
        ],
    )
    def kern(u_hbm, i_hbm, r_hbm, uY_hbm, iY_hbm, out_hbm,
             u_v, i_v, r_v, du_v, di_v, lu_v, li_v, sq_v, sem_u, sem_i):
        wid = lax.axis_index("s") * NC + lax.axis_index("c")
        base = wid * BPW

        pltpu.sync_copy(u_hbm.at[pl.ds(base, BPW)], u_v)
        pltpu.sync_copy(i_hbm.at[pl.ds(base, BPW)], i_v)
        pltpu.sync_copy(r_hbm.at[pl.ds(base, BPW)], r_v)

        def staged_row(ids):
            return lax.shift_left(
                lax.shift_right_logical(ids, TBLK_BITS),
                QROW_BITS) | (ids & (QROW - 1))

        @pl.loop(0, BPW, step=LANES)
        def _(k):
            du_v[pl.ds(k, LANES)] = staged_row(u_v[pl.ds(k, LANES)])
            di_v[pl.ds(k, LANES)] = staged_row(i_v[pl.ds(k, LANES)])

        sq_v[...] = jnp.zeros((LANES,), jnp.float32)

        for c in range(BPW // CHUNK):
            cbase = c * CHUNK
            cp_u = pltpu.async_copy(
                uY_hbm.at[du_v.at[pl.ds(cbase, CHUNK)]], lu_v, sem_u)
            cp_i = pltpu.async_copy(
                iY_hbm.at[di_v.at[pl.ds(cbase, CHUNK)]], li_v, sem_i)
            cp_u.wait()
            cp_i.wait()

            @pl.loop(0, CHUNK, step=LANES)
            def _(jb):
                rows = jb + lax.iota(jnp.int32, LANES)
                three = jnp.full((LANES,), 3, jnp.int32)
                zero = jnp.zeros((LANES,), jnp.int32)
                pu = lax.shift_right_logical(
                    u_v[pl.ds(cbase + jb, LANES)], QROW_BITS)
                pi = lax.shift_right_logical(
                    i_v[pl.ds(cbase + jb, LANES)], QROW_BITS)
                cu = (pu & three) * EMB_D
                ci = (pi & three) * EMB_D
                selu = (lax.shift_right_logical(pu, 2) & three) > zero
                seli = (lax.shift_right_logical(pi, 2) & three) > zero
                acc = jnp.zeros((LANES,), jnp.float32)
                for d in range(EMB_D):
                    ua = plsc.load_gather(lu_v, [rows, cu + d])
                    ia = plsc.load_gather(li_v, [rows, ci + d])
                    u_lo, u_hi = plsc.unpack(
                        plsc.bitcast(ua, jnp.bfloat16),
                        format=plsc.PackFormat.INTERLEAVED)
                    i_lo, i_hi = plsc.unpack(
                        plsc.bitcast(ia, jnp.bfloat16),
                        format=plsc.PackFormat.INTERLEAVED)
                    uv = jnp.where(selu, u_hi, u_lo)
                    iv = jnp.where(seli, i_hi, i_lo)
                    acc = acc + uv * iv
                err = r_v[pl.ds(cbase + jb, LANES)] - acc
                sq_v[...] = sq_v[...] + err * err

        pltpu.sync_copy(sq_v, out_hbm.at[wid])

    return kern(u_idx, i_idx, r, uY, iY)


def _tc_mean(partials):
    def body(p_ref, o_ref):
        o_ref[0, 0] = jnp.sum(p_ref[...]) * (1.0 / BATCH)

    out = pl.pallas_call(
        body,
        out_shape=jax.ShapeDtypeStruct((1, 1), jnp.float32),
        out_specs=pl.BlockSpec(memory_space=pltpu.SMEM),
    )(partials)
    return out[0, 0]


@jax.jit
def _mf_loss(interaction, uY, iY):
    u = interaction[:, 0].astype(jnp.int32)
    i = interaction[:, 1].astype(jnp.int32)
    r = interaction[:, 2].astype(jnp.float32)
    uYs = _tc_stage(uY.T)
    iYs = _tc_stage(iY.T)
    partials = _sc_partials(u, i, r, uYs, iYs)
    return _tc_mean(partials)


def kernel(interaction, uY, iY):
    return _mf_loss(interaction, uY, iY)


# speedup vs baseline: 5.3307x; 1.0193x over previous
import dataclasses
import functools

import jax
import jax.numpy as jnp
from jax import lax
from jax.experimental import pallas as pl
from jax.experimental.pallas import tpu as pltpu
from jax.experimental.pallas import tpu_sc as plsc

NU = 1000000
EMB_D = 32
BATCH = 16384
NC = 2
NS = 16
LANES = 16
NW = NC * NS
BPW = BATCH // NW
CHUNK = 128
TBLK_BITS = 17
TBLK = 1 << TBLK_BITS


GROUP = 8
GROUP_W = 128


QROW_BITS = TBLK_BITS - 3
QROW = TBLK // GROUP
NBLK = -(-NU // TBLK)
NROWS = NBLK * QROW


def _tc_stage(tY_t):
    def body(x_ref, o_ref):
        x = x_ref[...]
        xs = jnp.concatenate(
            [lax.slice(x, (0, QROW * p), (EMB_D, QROW * (p + 1)))
             for p in range(GROUP)], axis=0)
        vt = jnp.transpose(xs)
        bits = lax.bitcast_convert_type(vt, jnp.int32) + jnp.int32(0x8000)
        lo = lax.slice(bits, (0, 0), (QROW, GROUP_W))
        hi = lax.slice(bits, (0, GROUP_W), (QROW, 2 * GROUP_W))
        o_ref[...] = (lax.shift_right_logical(lo, 16)
                      | (hi & jnp.int32(-65536)))

    return pl.pallas_call(
        body,
        grid=(NBLK,),
        in_specs=[pl.BlockSpec((EMB_D, TBLK), lambda g: (0, g))],
        out_specs=pl.BlockSpec((QROW, GROUP_W), lambda g: (g, 0)),
        out_shape=jax.ShapeDtypeStruct((NROWS, GROUP_W), jnp.int32),
        compiler_params=pltpu.CompilerParams(
            dimension_semantics=("parallel",)),
    )(tY_t)


def _sc_partials(u_idx, i_idx, r, uY, iY):
    mesh = plsc.VectorSubcoreMesh(core_axis_name="c", subcore_axis_name="s")
    cp = pltpu.CompilerParams()
    if "needs_layout_passes" in pltpu.CompilerParams.__dataclass_fields__:
        cp = dataclasses.replace(cp, needs_layout_passes=False)

    @functools.partial(
        pl.kernel,
        mesh=mesh,
        compiler_params=cp,
        out_type=jax.ShapeDtypeStruct((NW, LANES), jnp.float32),
        scratch_types=[
            pltpu.VMEM((BPW,), jnp.int32),
            pltpu.VMEM((BPW,), jnp.int32),
            pltpu.VMEM((BPW,), jnp.float32),
            pltpu.VMEM((BPW,), jnp.int32),
            pltpu.VMEM((BPW,), jnp.int32),
            pltpu.VMEM((CHUNK, GROUP_W), jnp.int32),
            pltpu.VMEM((CHUNK, GROUP_W), jnp.int32),
            pltpu.VMEM((CHUNK, GROUP_W), jnp.int32),
            pltpu.VMEM((CHUNK, GROUP_W), jnp.int32),
            pltpu.VMEM((LANES,), jnp.float32),
            pltpu.SemaphoreType.DMA,
            pltpu.SemaphoreType.DMA,
            pltpu.SemaphoreType.DMA,
            pltpu.SemaphoreType.DMA,
        ],
    )
    def kern(u_hbm, i_hbm, r_hbm, uY_hbm, iY_hbm, out_hbm,
             u_v, i_v, r_v, du_v, di_v, lu_a, li_a, lu_b, li_b,
             sq_v, sem_u0, sem_i0, sem_u1, sem_i1):
        wid = lax.axis_index("s") * NC + lax.axis_index("c")
        base = wid * BPW

        pltpu.sync_copy(u_hbm.at[pl.ds(base, BPW)], u_v)
        pltpu.sync_copy(i_hbm.at[pl.ds(base, BPW)], i_v)
        pltpu.sync_copy(r_hbm.at[pl.ds(base, BPW)], r_v)

        def staged_row(ids):
            return lax.shift_left(
                lax.shift_right_logical(ids, TBLK_BITS),
                QROW_BITS) | (ids & (QROW - 1))

        @pl.loop(0, BPW, step=LANES)
        def _(k):
            du_v[pl.ds(k, LANES)] = staged_row(u_v[pl.ds(k, LANES)])
            di_v[pl.ds(k, LANES)] = staged_row(i_v[pl.ds(k, LANES)])

        sq_v[...] = jnp.zeros((LANES,), jnp.float32)

        bufs = [(lu_a, li_a, sem_u0, sem_i0), (lu_b, li_b, sem_u1, sem_i1)]
        nch = BPW // CHUNK

        def fire(c):
            lu_v, li_v, sem_u, sem_i = bufs[c % 2]
            cbase = c * CHUNK
            return (
                pltpu.async_copy(
                    uY_hbm.at[du_v.at[pl.ds(cbase, CHUNK)]], lu_v, sem_u),
                pltpu.async_copy(
                    iY_hbm.at[di_v.at[pl.ds(cbase, CHUNK)]], li_v, sem_i),
            )

        pending = fire(0)
        for c in range(nch):
            cbase = c * CHUNK
            lu_v, li_v = bufs[c % 2][0], bufs[c % 2][1]
            nxt = fire(c + 1) if c + 1 < nch else None
            pending[0].wait()
            pending[1].wait()
            pending = nxt

            @pl.loop(0, CHUNK, step=LANES)
            def _(jb):
                rows = jb + lax.iota(jnp.int32, LANES)
                three = jnp.full((LANES,), 3, jnp.int32)
                zero = jnp.zeros((LANES,), jnp.int32)
                pu = lax.shift_right_logical(
                    u_v[pl.ds(cbase + jb, LANES)], QROW_BITS)
                pi = lax.shift_right_logical(
                    i_v[pl.ds(cbase + jb, LANES)], QROW_BITS)
                cu = (pu & three) * EMB_D
                ci = (pi & three) * EMB_D
                selu = (lax.shift_right_logical(pu, 2) & three) > zero
                seli = (lax.shift_right_logical(pi, 2) & three) > zero
                acc = jnp.zeros((LANES,), jnp.float32)
                for d in range(EMB_D):
                    ua = plsc.load_gather(lu_v, [rows, cu + d])
                    ia = plsc.load_gather(li_v, [rows, ci + d])
                    u_lo, u_hi = plsc.unpack(
                        plsc.bitcast(ua, jnp.bfloat16),
                        format=plsc.PackFormat.INTERLEAVED)
                    i_lo, i_hi = plsc.unpack(
                        plsc.bitcast(ia, jnp.bfloat16),
                        format=plsc.PackFormat.INTERLEAVED)
                    uv = jnp.where(selu, u_hi, u_lo)
                    iv = jnp.where(seli, i_hi, i_lo)
                    acc = acc + uv * iv
                err = r_v[pl.ds(cbase + jb, LANES)] - acc
                sq_v[...] = sq_v[...] + err * err

        pltpu.sync_copy(sq_v, out_hbm.at[wid])

    return kern(u_idx, i_idx, r, uY, iY)


def _tc_mean(partials):
    def body(p_ref, o_ref):
        o_ref[0, 0] = jnp.sum(p_ref[...]) * (1.0 / BATCH)

    out = pl.pallas_call(
        body,
        out_shape=jax.ShapeDtypeStruct((1, 1), jnp.float32),
        out_specs=pl.BlockSpec(memory_space=pltpu.SMEM),
    )(partials)
    return out[0, 0]


@jax.jit
def _mf_loss(interaction, uY, iY):
    u = interaction[:, 0].astype(jnp.int32)
    i = interaction[:, 1].astype(jnp.int32)
    r = interaction[:, 2].astype(jnp.float32)
    uYs = _tc_stage(uY.T)
    iYs = _tc_stage(iY.T)
    partials = _sc_partials(u, i, r, uYs, iYs)
    return _tc_mean(partials)


def kernel(interaction, uY, iY):
    return _mf_loss(interaction, uY, iY)
